# baseline (speedup 1.0000x reference)
import functools

import jax
import jax.numpy as jnp
from jax import lax
from jax.experimental import pallas as pl
from jax.experimental.pallas import tpu as pltpu
from jax.experimental.pallas import tpu_sc as plsc

B = 16384
D = 32
WIDE = 128
RPW_TAB = WIDE // D
NC = 2
NS = 16
NW = NC * NS
BPW = B // NW
CHUNK = 128
NCH = BPW // CHUNK
TC_BS = 2048
FMT_COLS = 2048
FMT_ROWS = FMT_COLS // RPW_TAB


def _tc_format(tt, eye, n_rows):
    grid = (n_rows + FMT_COLS - 1) // FMT_COLS

    def body(t_ref, e_ref, o_ref):
        o_ref[...] = jnp.concatenate(
            [lax.dot_general(t_ref[:, p * FMT_ROWS:(p + 1) * FMT_ROWS],
                             e_ref[...],
                             (((0,), (0,)), ((), ())),
                             preferred_element_type=jnp.float32)
             for p in range(RPW_TAB)], axis=1)

    return pl.pallas_call(
        body,
        grid=(grid,),
        in_specs=[pl.BlockSpec((D, FMT_COLS), lambda g: (0, g)),
                  pl.BlockSpec((D, D), lambda g: (0, 0))],
        out_specs=pl.BlockSpec((FMT_ROWS, WIDE), lambda g: (g, 0)),
        out_shape=jax.ShapeDtypeStruct((grid * FMT_ROWS, WIDE), jnp.float32),
    )(tt, eye)


def _sc_gather_wide(uidx, iidx, utab_w, itab_w):
    mesh = plsc.VectorSubcoreMesh(
        core_axis_name="c", subcore_axis_name="s",
        num_cores=NC, num_subcores=NS)

    @functools.partial(
        pl.kernel, mesh=mesh,
        compiler_params=pltpu.CompilerParams(use_tc_tiling_on_sc=True),
        out_type=(jax.ShapeDtypeStruct((B, WIDE), jnp.float32),
                  jax.ShapeDtypeStruct((B, WIDE), jnp.float32)),
        scratch_types=[
            pltpu.VMEM((NCH, CHUNK), jnp.int32),
            pltpu.VMEM((NCH, CHUNK), jnp.int32),
            pltpu.VMEM((2, CHUNK, WIDE), jnp.float32),
            pltpu.VMEM((2, CHUNK, WIDE), jnp.float32),
            pltpu.SemaphoreType.DMA,
        ],
    )
    def k(uidx_hbm, iidx_hbm, utab, itab, uout, iout,
          uidx_v, iidx_v, ubuf, ibuf, sem):
        wid = lax.axis_index("s") * NC + lax.axis_index("c")
        base = wid * BPW
        pltpu.sync_copy(uidx_hbm.at[wid], uidx_v)
        pltpu.sync_copy(iidx_hbm.at[wid], iidx_v)
        gathers = []
        for j in range(NCH):
            gathers.append((
                pltpu.async_copy(utab.at[uidx_v.at[j]], ubuf.at[j % 2], sem),
                pltpu.async_copy(itab.at[iidx_v.at[j]], ibuf.at[j % 2], sem),
            ))
            if j > 0:
                gu, gi = gathers[j - 1]
                gu.wait()
                gi.wait()
                off = base + (j - 1) * CHUNK
                pltpu.sync_copy(ubuf.at[(j - 1) % 2],
                                uout.at[pl.ds(off, CHUNK)])
                pltpu.sync_copy(ibuf.at[(j - 1) % 2],
                                iout.at[pl.ds(off, CHUNK)])
        gu, gi = gathers[NCH - 1]
        gu.wait()
        gi.wait()
        off = base + (NCH - 1) * CHUNK
        pltpu.sync_copy(ubuf.at[(NCH - 1) % 2], uout.at[pl.ds(off, CHUNK)])
        pltpu.sync_copy(ibuf.at[(NCH - 1) % 2], iout.at[pl.ds(off, CHUNK)])

    return k(uidx, iidx, utab_w, itab_w)


def _tc_combine(uwide, iwide, usub, isub, wt, b2):
    def body(uw_ref, iw_ref, us_ref, is_ref, wt_ref, b_ref, o_ref):
        us = us_ref[...]
        isv = is_ref[...]
        u = uw_ref[:, 0:D]
        i = iw_ref[:, 0:D]
        for m in range(1, RPW_TAB):
            sl = slice(m * D, (m + 1) * D)
            u = jnp.where(us == m, uw_ref[:, sl], u)
            i = jnp.where(isv == m, iw_ref[:, sl], i)
        proj = jnp.dot(u, wt_ref[...],
                       preferred_element_type=jnp.float32) + b_ref[...]
        o_ref[...] = jnp.sum(proj * i, axis=1)

    grid = B // TC_BS
    return pl.pallas_call(
        body,
        grid=(grid,),
        in_specs=[
            pl.BlockSpec((TC_BS, WIDE), lambda g: (g, 0)),
            pl.BlockSpec((TC_BS, WIDE), lambda g: (g, 0)),
            pl.BlockSpec((TC_BS, 1), lambda g: (g, 0)),
            pl.BlockSpec((TC_BS, 1), lambda g: (g, 0)),
            pl.BlockSpec((D, D), lambda g: (0, 0)),
            pl.BlockSpec((1, D), lambda g: (0, 0)),
        ],
        out_specs=pl.BlockSpec((TC_BS,), lambda g: (g,)),
        out_shape=jax.ShapeDtypeStruct((B,), jnp.float32),
    )(uwide, iwide, usub, isub, wt, b2)


def kernel(x, uid_table, iid_table, W, b):
    eye = jnp.eye(D, dtype=jnp.float32)
    utab_w = _tc_format(uid_table.T, eye, uid_table.shape[0])
    itab_w = _tc_format(iid_table.T, eye, iid_table.shape[0])
    ru, ri = x[:, 0], x[:, 1]
    uidx = (FMT_ROWS * (ru // FMT_COLS) + ru % FMT_ROWS).reshape(NW, NCH, CHUNK)
    iidx = (FMT_ROWS * (ri // FMT_COLS) + ri % FMT_ROWS).reshape(NW, NCH, CHUNK)
    usub = ((ru // FMT_ROWS) & (RPW_TAB - 1)).reshape(B, 1)
    isub = ((ri // FMT_ROWS) & (RPW_TAB - 1)).reshape(B, 1)
    uwide, iwide = _sc_gather_wide(uidx, iidx, utab_w, itab_w)
    return _tc_combine(uwide, iwide, usub, isub, W.T, b.reshape(1, D))


# speedup vs baseline: 1.5539x; 1.5539x over previous
import functools

import jax
import jax.numpy as jnp
from jax import lax
from jax.experimental import pallas as pl
from jax.experimental.pallas import tpu as pltpu
from jax.experimental.pallas import tpu_sc as plsc

B = 16384
D = 32
WIDE = 128
RPW_TAB = WIDE // D
NC = 2
NS = 16
NW = NC * NS
BPW = B // NW
CHUNK = 128
NCH = BPW // CHUNK
TC_BS = 2048
FMT_COLS = 8192
FMT_ROWS = FMT_COLS // RPW_TAB


def _tc_format(tt, eye, n_rows):
    grid = (n_rows + FMT_COLS - 1) // FMT_COLS

    def body(t_ref, e_ref, o_ref):
        o_ref[...] = jnp.concatenate(
            [lax.dot_general(t_ref[:, p * FMT_ROWS:(p + 1) * FMT_ROWS],
                             e_ref[...],
                             (((0,), (0,)), ((), ())),
                             preferred_element_type=jnp.float32)
             for p in range(RPW_TAB)], axis=1)

    return pl.pallas_call(
        body,
        grid=(grid,),
        in_specs=[pl.BlockSpec((D, FMT_COLS), lambda g: (0, g)),
                  pl.BlockSpec((D, D), lambda g: (0, 0))],
        out_specs=pl.BlockSpec((FMT_ROWS, WIDE), lambda g: (g, 0)),
        out_shape=jax.ShapeDtypeStruct((grid * FMT_ROWS, WIDE), jnp.float32),
    )(tt, eye)


def _sc_gather_wide(uidx, iidx, utab_w, itab_w):
    mesh = plsc.VectorSubcoreMesh(
        core_axis_name="c", subcore_axis_name="s",
        num_cores=NC, num_subcores=NS)

    @functools.partial(
        pl.kernel, mesh=mesh,
        compiler_params=pltpu.CompilerParams(use_tc_tiling_on_sc=True),
        out_type=(jax.ShapeDtypeStruct((B, WIDE), jnp.float32),
                  jax.ShapeDtypeStruct((B, WIDE), jnp.float32)),
        scratch_types=[
            pltpu.VMEM((NCH, CHUNK), jnp.int32),
            pltpu.VMEM((NCH, CHUNK), jnp.int32),
            pltpu.VMEM((2, CHUNK, WIDE), jnp.float32),
            pltpu.VMEM((2, CHUNK, WIDE), jnp.float32),
            pltpu.SemaphoreType.DMA,
        ],
    )
    def k(uidx_hbm, iidx_hbm, utab, itab, uout, iout,
          uidx_v, iidx_v, ubuf, ibuf, sem):
        wid = lax.axis_index("s") * NC + lax.axis_index("c")
        base = wid * BPW
        pltpu.sync_copy(uidx_hbm.at[wid], uidx_v)
        pltpu.sync_copy(iidx_hbm.at[wid], iidx_v)
        gathers = []
        for j in range(NCH):
            gathers.append((
                pltpu.async_copy(utab.at[uidx_v.at[j]], ubuf.at[j % 2], sem),
                pltpu.async_copy(itab.at[iidx_v.at[j]], ibuf.at[j % 2], sem),
            ))
            if j > 0:
                gu, gi = gathers[j - 1]
                gu.wait()
                gi.wait()
                off = base + (j - 1) * CHUNK
                pltpu.sync_copy(ubuf.at[(j - 1) % 2],
                                uout.at[pl.ds(off, CHUNK)])
                pltpu.sync_copy(ibuf.at[(j - 1) % 2],
                                iout.at[pl.ds(off, CHUNK)])
        gu, gi = gathers[NCH - 1]
        gu.wait()
        gi.wait()
        off = base + (NCH - 1) * CHUNK
        pltpu.sync_copy(ubuf.at[(NCH - 1) % 2], uout.at[pl.ds(off, CHUNK)])
        pltpu.sync_copy(ibuf.at[(NCH - 1) % 2], iout.at[pl.ds(off, CHUNK)])

    return k(uidx, iidx, utab_w, itab_w)


def _tc_combine(uwide, iwide, usub, isub, wt, b2):
    def body(uw_ref, iw_ref, us_ref, is_ref, wt_ref, b_ref, o_ref):
        us = us_ref[...]
        isv = is_ref[...]
        u = uw_ref[:, 0:D]
        i = iw_ref[:, 0:D]
        for m in range(1, RPW_TAB):
            sl = slice(m * D, (m + 1) * D)
            u = jnp.where(us == m, uw_ref[:, sl], u)
            i = jnp.where(isv == m, iw_ref[:, sl], i)
        proj = jnp.dot(u, wt_ref[...],
                       preferred_element_type=jnp.float32) + b_ref[...]
        o_ref[...] = jnp.sum(proj * i, axis=1)

    grid = B // TC_BS
    return pl.pallas_call(
        body,
        grid=(grid,),
        in_specs=[
            pl.BlockSpec((TC_BS, WIDE), lambda g: (g, 0)),
            pl.BlockSpec((TC_BS, WIDE), lambda g: (g, 0)),
            pl.BlockSpec((TC_BS, 1), lambda g: (g, 0)),
            pl.BlockSpec((TC_BS, 1), lambda g: (g, 0)),
            pl.BlockSpec((D, D), lambda g: (0, 0)),
            pl.BlockSpec((1, D), lambda g: (0, 0)),
        ],
        out_specs=pl.BlockSpec((TC_BS,), lambda g: (g,)),
        out_shape=jax.ShapeDtypeStruct((B,), jnp.float32),
    )(uwide, iwide, usub, isub, wt, b2)


def kernel(x, uid_table, iid_table, W, b):
    eye = jnp.eye(D, dtype=jnp.float32)
    utab_w = _tc_format(uid_table.T, eye, uid_table.shape[0])
    itab_w = _tc_format(iid_table.T, eye, iid_table.shape[0])
    ru, ri = x[:, 0], x[:, 1]
    uidx = (FMT_ROWS * (ru // FMT_COLS) + ru % FMT_ROWS).reshape(NW, NCH, CHUNK)
    iidx = (FMT_ROWS * (ri // FMT_COLS) + ri % FMT_ROWS).reshape(NW, NCH, CHUNK)
    usub = ((ru // FMT_ROWS) & (RPW_TAB - 1)).reshape(B, 1)
    isub = ((ri // FMT_ROWS) & (RPW_TAB - 1)).reshape(B, 1)
    uwide, iwide = _sc_gather_wide(uidx, iidx, utab_w, itab_w)
    return _tc_combine(uwide, iwide, usub, isub, W.T, b.reshape(1, D))


# speedup vs baseline: 1.5783x; 1.0157x over previous
import functools

import jax
import jax.numpy as jnp
from jax import lax
from jax.experimental import pallas as pl
from jax.experimental.pallas import tpu as pltpu
from jax.experimental.pallas import tpu_sc as plsc

B = 16384
D = 32
WIDE = 128
RPW_TAB = WIDE // D
NC = 2
NS = 16
NW = NC * NS
BPW = B // NW
CHUNK = 128
NCH = BPW // CHUNK
TC_BS = 2048
FMT_COLS = 16384
FMT_ROWS = FMT_COLS // RPW_TAB


def _tc_format(tt, eye, n_rows):
    grid = (n_rows + FMT_COLS - 1) // FMT_COLS

    def body(t_ref, e_ref, o_ref):
        o_ref[...] = jnp.concatenate(
            [lax.dot_general(t_ref[:, p * FMT_ROWS:(p + 1) * FMT_ROWS],
                             e_ref[...],
                             (((0,), (0,)), ((), ())),
                             preferred_element_type=jnp.float32)
             for p in range(RPW_TAB)], axis=1)

    return pl.pallas_call(
        body,
        grid=(grid,),
        in_specs=[pl.BlockSpec((D, FMT_COLS), lambda g: (0, g)),
                  pl.BlockSpec((D, D), lambda g: (0, 0))],
        out_specs=pl.BlockSpec((FMT_ROWS, WIDE), lambda g: (g, 0)),
        out_shape=jax.ShapeDtypeStruct((grid * FMT_ROWS, WIDE), jnp.float32),
    )(tt, eye)


def _sc_gather_wide(uidx, iidx, utab_w, itab_w):
    mesh = plsc.VectorSubcoreMesh(
        core_axis_name="c", subcore_axis_name="s",
        num_cores=NC, num_subcores=NS)

    @functools.partial(
        pl.kernel, mesh=mesh,
        compiler_params=pltpu.CompilerParams(use_tc_tiling_on_sc=True),
        out_type=(jax.ShapeDtypeStruct((B, WIDE), jnp.float32),
                  jax.ShapeDtypeStruct((B, WIDE), jnp.float32)),
        scratch_types=[
            pltpu.VMEM((NCH, CHUNK), jnp.int32),
            pltpu.VMEM((NCH, CHUNK), jnp.int32),
            pltpu.VMEM((2, CHUNK, WIDE), jnp.float32),
            pltpu.VMEM((2, CHUNK, WIDE), jnp.float32),
            pltpu.SemaphoreType.DMA,
        ],
    )
    def k(uidx_hbm, iidx_hbm, utab, itab, uout, iout,
          uidx_v, iidx_v, ubuf, ibuf, sem):
        wid = lax.axis_index("s") * NC + lax.axis_index("c")
        base = wid * BPW
        pltpu.sync_copy(uidx_hbm.at[wid], uidx_v)
        pltpu.sync_copy(iidx_hbm.at[wid], iidx_v)
        gathers = []
        for j in range(NCH):
            gathers.append((
                pltpu.async_copy(utab.at[uidx_v.at[j]], ubuf.at[j % 2], sem),
                pltpu.async_copy(itab.at[iidx_v.at[j]], ibuf.at[j % 2], sem),
            ))
            if j > 0:
                gu, gi = gathers[j - 1]
                gu.wait()
                gi.wait()
                off = base + (j - 1) * CHUNK
                pltpu.sync_copy(ubuf.at[(j - 1) % 2],
                                uout.at[pl.ds(off, CHUNK)])
                pltpu.sync_copy(ibuf.at[(j - 1) % 2],
                                iout.at[pl.ds(off, CHUNK)])
        gu, gi = gathers[NCH - 1]
        gu.wait()
        gi.wait()
        off = base + (NCH - 1) * CHUNK
        pltpu.sync_copy(ubuf.at[(NCH - 1) % 2], uout.at[pl.ds(off, CHUNK)])
        pltpu.sync_copy(ibuf.at[(NCH - 1) % 2], iout.at[pl.ds(off, CHUNK)])

    return k(uidx, iidx, utab_w, itab_w)


def _tc_combine(uwide, iwide, usub, isub, wt, b2):
    def body(uw_ref, iw_ref, us_ref, is_ref, wt_ref, b_ref, o_ref):
        us = us_ref[...]
        isv = is_ref[...]
        u = uw_ref[:, 0:D]
        i = iw_ref[:, 0:D]
        for m in range(1, RPW_TAB):
            sl = slice(m * D, (m + 1) * D)
            u = jnp.where(us == m, uw_ref[:, sl], u)
            i = jnp.where(isv == m, iw_ref[:, sl], i)
        proj = jnp.dot(u, wt_ref[...],
                       preferred_element_type=jnp.float32) + b_ref[...]
        o_ref[...] = jnp.sum(proj * i, axis=1)

    grid = B // TC_BS
    return pl.pallas_call(
        body,
        grid=(grid,),
        in_specs=[
            pl.BlockSpec((TC_BS, WIDE), lambda g: (g, 0)),
            pl.BlockSpec((TC_BS, WIDE), lambda g: (g, 0)),
            pl.BlockSpec((TC_BS, 1), lambda g: (g, 0)),
            pl.BlockSpec((TC_BS, 1), lambda g: (g, 0)),
            pl.BlockSpec((D, D), lambda g: (0, 0)),
            pl.BlockSpec((1, D), lambda g: (0, 0)),
        ],
        out_specs=pl.BlockSpec((TC_BS,), lambda g: (g,)),
        out_shape=jax.ShapeDtypeStruct((B,), jnp.float32),
    )(uwide, iwide, usub, isub, wt, b2)


def kernel(x, uid_table, iid_table, W, b):
    eye = jnp.eye(D, dtype=jnp.float32)
    utab_w = _tc_format(uid_table.T, eye, uid_table.shape[0])
    itab_w = _tc_format(iid_table.T, eye, iid_table.shape[0])
    ru, ri = x[:, 0], x[:, 1]
    uidx = (FMT_ROWS * (ru // FMT_COLS) + ru % FMT_ROWS).reshape(NW, NCH, CHUNK)
    iidx = (FMT_ROWS * (ri // FMT_COLS) + ri % FMT_ROWS).reshape(NW, NCH, CHUNK)
    usub = ((ru // FMT_ROWS) & (RPW_TAB - 1)).reshape(B, 1)
    isub = ((ri // FMT_ROWS) & (RPW_TAB - 1)).reshape(B, 1)
    uwide, iwide = _sc_gather_wide(uidx, iidx, utab_w, itab_w)
    return _tc_combine(uwide, iwide, usub, isub, W.T, b.reshape(1, D))
